# Initial kernel scaffold; baseline (speedup 1.0000x reference)
import jax
import jax.numpy as jnp
from jax.experimental import pallas as pl


def kernel(org_obs, node_embeddings, substation_embeddings, sub_choice, W_proj, b_proj, W_gat, a_src, a_dst, b_gat):
    raise NotImplementedError("write your pallas kernel here")


# speedup vs baseline: 702.1373x; 702.1373x over previous
import functools

import jax
import jax.numpy as jnp
from jax import lax
from jax.experimental import pallas as pl
from jax.experimental.pallas import tpu as pltpu
from jax.experimental.pallas import tpu_sc as plsc

B = 1024
N = 256
S = 16
H = 128
K = 16
L = 16
NC = 2
NS = 16
NW = NC * NS
BPW = B // NW
NCH = H // L


def _lrelu(x):
    return jnp.where(x >= 0, x, 0.2 * x)


@functools.partial(
    pl.kernel,
    out_type=jax.ShapeDtypeStruct((B, K), jnp.float32),
    mesh=plsc.VectorSubcoreMesh(core_axis_name="c", subcore_axis_name="s"),
    compiler_params=pltpu.CompilerParams(needs_layout_passes=False),
    scratch_types=[
        pltpu.VMEM((BPW,), jnp.int32),
        pltpu.VMEM((BPW,), jnp.int32),
        pltpu.VMEM((BPW, K * H), jnp.float32),
        pltpu.VMEM((BPW, H), jnp.float32),
        pltpu.VMEM((BPW, H), jnp.float32),
        pltpu.VMEM((3 * H,), jnp.float32),
        pltpu.VMEM((L,), jnp.float32),
        pltpu.VMEM((BPW, K), jnp.float32),
        pltpu.SemaphoreType.DMA,
    ],
)
def _sc_fwd(nodes_hbm, subs_hbm, obs_hbm, subc_hbm, w_hbm, p_hbm, out_hbm,
            idx_v, subc_v, nodes_v, subs_v, obs_v, wv_v, par_v,
            out_v, sem):
    wid = lax.axis_index("s") * NC + lax.axis_index("c")
    base = wid * BPW

    pltpu.sync_copy(subc_hbm.at[pl.ds(base, BPW)], subc_v)
    pltpu.sync_copy(obs_hbm.at[pl.ds(base, BPW)], obs_v)
    pltpu.sync_copy(w_hbm, wv_v)
    pltpu.sync_copy(p_hbm, par_v)

    iota = lax.iota(jnp.int32, L)
    for half in range(BPW // L):
        sub = subc_v[pl.ds(half * L, L)]
        row = (base + half * L + iota) * S + sub
        idx_v[pl.ds(half * L, L)] = row

    cp_n = pltpu.async_copy(nodes_hbm.at[idx_v], nodes_v, sem)
    cp_s = pltpu.async_copy(subs_hbm.at[idx_v], subs_v, sem)
    cp_n.wait()
    cp_s.wait()

    par = par_v[...]
    c0 = par[0]
    a_src = par[1]
    a_dst = par[2]
    b_gat = par[3]
    lane = lax.iota(jnp.int32, L)

    def body(s, carry):
        acc = (obs_v[s, pl.ds(0, L)] * wv_v[pl.ds(0, L)]
               + subs_v[s, pl.ds(0, L)] * wv_v[pl.ds(H, L)])
        for c in range(1, NCH):
            acc = acc + obs_v[s, pl.ds(c * L, L)] * wv_v[pl.ds(c * L, L)]
            acc = acc + subs_v[s, pl.ds(c * L, L)] * wv_v[pl.ds(H + c * L, L)]
        hs = jnp.sum(acc) + c0

        hvec = None
        for k in range(K):
            a2 = nodes_v[s, pl.ds(k * H, L)] * wv_v[pl.ds(2 * H, L)]
            for c in range(1, NCH):
                a2 = a2 + (nodes_v[s, pl.ds(k * H + c * L, L)]
                           * wv_v[pl.ds(2 * H + c * L, L)])
            hk = jnp.full((L,), jnp.sum(a2) + hs)
            hvec = hk if hvec is None else jnp.where(lane == k, hk, hvec)

        svec = hvec * a_src
        dvec = hvec * a_dst
        mvec = _lrelu(jnp.max(svec) + dvec)

        den = None
        num = None
        for i in range(K):
            e = _lrelu(svec[i] + dvec)
            w = jnp.exp(e - mvec)
            den = w if den is None else den + w
            num = w * hvec[i] if num is None else num + w * hvec[i]
        out_v[s, :] = num / den + b_gat
        return carry

    lax.fori_loop(0, BPW, body, 0)
    pltpu.sync_copy(out_v, out_hbm.at[pl.ds(base, BPW)])


def kernel(org_obs, node_embeddings, substation_embeddings, sub_choice,
           W_proj, b_proj, W_gat, a_src, a_dst, b_gat):
    w = W_gat[:, 0]
    w1, w2, w3 = w[:H], w[H:2 * H], w[2 * H:]
    v = W_proj @ w1
    c0 = jnp.dot(b_proj, w1)
    wvec = jnp.concatenate([v, w2, w3]).astype(jnp.float32)
    params = jnp.zeros((L,), jnp.float32)
    params = params.at[0].set(c0)
    params = params.at[1].set(a_src[0])
    params = params.at[2].set(a_dst[0])
    params = params.at[3].set(b_gat[0])

    nodes_flat = node_embeddings.reshape(B * S, K * H)
    subs_flat = substation_embeddings.reshape(B * S, H)
    subc = sub_choice.reshape(B).astype(jnp.int32)

    out = _sc_fwd(nodes_flat, subs_flat, org_obs, subc, wvec, params)
    return (out.reshape(B * K, 1), sub_choice)


# speedup vs baseline: 3011.4725x; 4.2890x over previous
import functools

import jax
import jax.numpy as jnp
from jax import lax
from jax.experimental import pallas as pl
from jax.experimental.pallas import tpu as pltpu
from jax.experimental.pallas import tpu_sc as plsc

B = 1024
N = 256
S = 16
H = 128
K = 16
L = 16
NC = 2
NS = 16
NW = NC * NS
BPW = B // NW
NCH = H // L


def _lrelu(x):
    return jnp.where(x >= 0, x, 0.2 * x)


@functools.partial(
    pl.kernel,
    out_type=jax.ShapeDtypeStruct((B, K), jnp.float32),
    mesh=plsc.VectorSubcoreMesh(core_axis_name="c", subcore_axis_name="s"),
    compiler_params=pltpu.CompilerParams(needs_layout_passes=False),
    scratch_types=[
        pltpu.VMEM((BPW,), jnp.int32),
        pltpu.VMEM((4, BPW * K // 4), jnp.int32),
        pltpu.VMEM((BPW,), jnp.int32),
        pltpu.VMEM((BPW * K, H), jnp.float32),
        pltpu.VMEM((BPW, H), jnp.float32),
        pltpu.VMEM((BPW, H), jnp.float32),
        pltpu.VMEM((3 * H,), jnp.float32),
        pltpu.VMEM((L,), jnp.float32),
        pltpu.VMEM((BPW, K), jnp.float32),
        pltpu.SemaphoreType.DMA,
    ],
)
def _sc_fwd(nodes_hbm, subs_hbm, obs_hbm, subc_hbm, w_hbm, p_hbm, out_hbm,
            idx_v, nidx_v, subc_v, nodes_v, subs_v, obs_v, wv_v, par_v,
            out_v, sem):
    wid = lax.axis_index("s") * NC + lax.axis_index("c")
    base = wid * BPW

    pltpu.sync_copy(subc_hbm.at[pl.ds(base, BPW)], subc_v)
    pltpu.sync_copy(obs_hbm.at[pl.ds(base, BPW)], obs_v)
    pltpu.sync_copy(w_hbm, wv_v)
    pltpu.sync_copy(p_hbm, par_v)

    iota = lax.iota(jnp.int32, L)
    subh = [subc_v[pl.ds(0, L)], subc_v[pl.ds(L, L)]]
    for half in range(BPW // L):
        row = (base + half * L + iota) * S + subh[half]
        idx_v[pl.ds(half * L, L)] = row
    rows_per_q = BPW * K // 4
    for j in range(BPW):
        sub_j = subh[j // L][j % L]
        nbase = (base + j) * N + sub_j * K
        q, r = divmod(j * K, rows_per_q)
        nidx_v[q, pl.ds(r, K)] = nbase + iota

    cps = [
        pltpu.async_copy(
            nodes_hbm.at[nidx_v.at[q]],
            nodes_v.at[pl.ds(q * rows_per_q, rows_per_q), :],
            sem,
        )
        for q in range(4)
    ]
    cp_s = pltpu.async_copy(subs_hbm.at[idx_v], subs_v, sem)
    for cp in cps:
        cp.wait()
    cp_s.wait()

    par = par_v[...]
    c0 = par[0]
    a_src = par[1]
    a_dst = par[2]
    b_gat = par[3]
    lane = lax.iota(jnp.int32, L)

    def body(s, carry):
        acc = (obs_v[s, pl.ds(0, L)] * wv_v[pl.ds(0, L)]
               + subs_v[s, pl.ds(0, L)] * wv_v[pl.ds(H, L)])
        for c in range(1, NCH):
            acc = acc + obs_v[s, pl.ds(c * L, L)] * wv_v[pl.ds(c * L, L)]
            acc = acc + subs_v[s, pl.ds(c * L, L)] * wv_v[pl.ds(H + c * L, L)]
        hs = jnp.sum(acc) + c0

        hvec = None
        for k in range(K):
            a2 = nodes_v[s * K + k, pl.ds(0, L)] * wv_v[pl.ds(2 * H, L)]
            for c in range(1, NCH):
                a2 = a2 + (nodes_v[s * K + k, pl.ds(c * L, L)]
                           * wv_v[pl.ds(2 * H + c * L, L)])
            hk = jnp.full((L,), jnp.sum(a2) + hs)
            hvec = hk if hvec is None else jnp.where(lane == k, hk, hvec)

        svec = hvec * a_src
        dvec = hvec * a_dst
        mvec = _lrelu(jnp.max(svec) + dvec)

        den = None
        num = None
        for i in range(K):
            e = _lrelu(svec[i] + dvec)
            w = jnp.exp(e - mvec)
            den = w if den is None else den + w
            num = w * hvec[i] if num is None else num + w * hvec[i]
        out_v[s, :] = num / den + b_gat
        return carry

    lax.fori_loop(0, BPW, body, 0)
    pltpu.sync_copy(out_v, out_hbm.at[pl.ds(base, BPW)])


def kernel(org_obs, node_embeddings, substation_embeddings, sub_choice,
           W_proj, b_proj, W_gat, a_src, a_dst, b_gat):
    w = W_gat[:, 0]
    w1, w2, w3 = w[:H], w[H:2 * H], w[2 * H:]
    v = W_proj @ w1
    c0 = jnp.dot(b_proj, w1)
    wvec = jnp.concatenate([v, w2, w3]).astype(jnp.float32)
    params = jnp.zeros((L,), jnp.float32)
    params = params.at[0].set(c0)
    params = params.at[1].set(a_src[0])
    params = params.at[2].set(a_dst[0])
    params = params.at[3].set(b_gat[0])

    nodes_flat = node_embeddings.reshape(B * N, H)
    subs_flat = substation_embeddings.reshape(B * S, H)
    subc = sub_choice.reshape(B).astype(jnp.int32)

    out = _sc_fwd(nodes_flat, subs_flat, org_obs, subc, wvec, params)
    return (out.reshape(B * K, 1), sub_choice)
